# Initial kernel scaffold; baseline (speedup 1.0000x reference)
import jax
import jax.numpy as jnp
from jax.experimental import pallas as pl


def kernel(feat, edge_index, etypes, norm, bases, coeff, attn_weight, attn_vec, m_rel, h_bias, ln_gamma, ln_beta, loop_weight):
    raise NotImplementedError("write your pallas kernel here")


# speedup vs baseline: 17.2726x; 17.2726x over previous
import jax
import jax.numpy as jnp
from jax import lax
from jax.experimental import pallas as pl
from jax.experimental.pallas import tpu as pltpu
from jax.experimental.pallas import tpu_sc as plsc

N = 10000
E = 320000
IN = 128
OUT = 128
R = 8
B = 4

NP = 10240
NC = 2
NS = 16
NW = NC * NS
EPT = E // NW
C = 80
NCHUNK = EPT // C
RPT = NP // NS
ESR = NP // 128
EST = 8
TBLK = 256


def _full(shape):
    return pl.BlockSpec(shape, lambda: tuple(0 for _ in shape))


def _t0_body(bases_ref, aw_ref, av_ref, mrel_ref, coeff_ref, w2_ref, u13_ref,
             s2_ref):
    av = av_ref[...]
    v13 = jnp.concatenate([av[0:IN], av[2 * IN:3 * IN]], axis=1)
    u13_ref[...] = jnp.dot(aw_ref[...], v13, preferred_element_type=jnp.float32)
    u2 = jnp.dot(aw_ref[...], av[IN:2 * IN], preferred_element_type=jnp.float32)
    s2_ref[...] = jnp.dot(mrel_ref[...], u2, preferred_element_type=jnp.float32)
    for r in range(R):
        acc = jnp.zeros((IN, OUT), jnp.float32)
        for b in range(B):
            acc = acc + coeff_ref[r, b] * bases_ref[b * IN:(b + 1) * IN, :]
        w2_ref[:, r * OUT:(r + 1) * OUT] = acc


def _t0(bases_flat, attn_weight, attn_vec, m_rel, coeff):
    return pl.pallas_call(
        _t0_body,
        in_specs=[
            _full((B * IN, OUT)),
            _full((IN, IN)),
            _full((3 * IN, 1)),
            _full((R, IN)),
            pl.BlockSpec(memory_space=pltpu.SMEM),
        ],
        out_specs=(
            _full((IN, R * OUT)),
            _full((IN, 2)),
            _full((R, 1)),
        ),
        out_shape=(
            jax.ShapeDtypeStruct((IN, R * OUT), jnp.float32),
            jax.ShapeDtypeStruct((IN, 2), jnp.float32),
            jax.ShapeDtypeStruct((R, 1), jnp.float32),
        ),
    )(bases_flat, attn_weight, attn_vec, m_rel, coeff)


def _t1_body(f_ref, w2_ref, u13_ref, q_ref, s13_ref):
    f = f_ref[...]
    q_ref[...] = jnp.dot(f, w2_ref[...], preferred_element_type=jnp.float32)
    s13_ref[...] = jnp.dot(f, u13_ref[...], preferred_element_type=jnp.float32)


def _t1(feat_p, w2, u13):
    nblk = NP // TBLK
    return pl.pallas_call(
        _t1_body,
        grid=(nblk,),
        in_specs=[
            pl.BlockSpec((TBLK, IN), lambda i: (i, 0)),
            pl.BlockSpec((IN, R * OUT), lambda i: (0, 0)),
            pl.BlockSpec((IN, 2), lambda i: (0, 0)),
        ],
        out_specs=(
            pl.BlockSpec((TBLK, R * OUT), lambda i: (i, 0)),
            pl.BlockSpec((TBLK, 2), lambda i: (i, 0)),
        ),
        out_shape=(
            jax.ShapeDtypeStruct((NP, R * OUT), jnp.float32),
            jax.ShapeDtypeStruct((NP, 2), jnp.float32),
        ),
    )(feat_p, w2, u13)


def _sc_body(src_hbm, dst_hbm, et_hbm, nrm_hbm, s1_hbm, s3_hbm, s2_hbm, q_hbm,
             h_out, es_out, h_sp, s1_v, s3_v, s2_v, esum_v, src_v,
             dst_v, et_v, nrm_v, gidx_v, w_v, ee_v, rows_g, acc_v,
             tmp_v, sem):
    cid = lax.axis_index("c")
    sid = lax.axis_index("s")
    wid = cid * NS + sid
    row0 = sid * RPT

    zeros16 = jnp.zeros((16,), jnp.float32)

    def _zb(i, c):
        for k in range(OUT // 16):
            rows_g[i, pl.ds(k * 16, 16)] = zeros16
        return c

    lax.fori_loop(0, C, _zb, 0)

    def _zc(k, c):
        pltpu.sync_copy(rows_g, h_sp.at[pl.ds(row0 + k * C, C)])
        return c

    lax.fori_loop(0, RPT // C, _zc, 0)

    def _ze(i, c):
        for k in range(128 // 16):
            esum_v[i, pl.ds(k * 16, 16)] = zeros16
        return c

    lax.fori_loop(0, ESR, _ze, 0)

    pltpu.sync_copy(s1_hbm, s1_v)
    pltpu.sync_copy(s3_hbm, s3_v)
    pltpu.sync_copy(s2_hbm, s2_v)
    plsc.subcore_barrier()

    def _chunk(g, c):
        base = wid * EPT + g * C
        pltpu.sync_copy(src_hbm.at[pl.ds(base, C)], src_v)
        pltpu.sync_copy(dst_hbm.at[pl.ds(base, C)], dst_v)
        pltpu.sync_copy(et_hbm.at[pl.ds(base, C)], et_v)
        pltpu.sync_copy(nrm_hbm.at[pl.ds(base, C)], nrm_v)
        for j in range(C // 16):
            sl = pl.ds(j * 16, 16)
            s = src_v[sl]
            d = dst_v[sl]
            t = et_v[sl]
            a1 = plsc.load_gather(s1_v, [s])
            a3 = plsc.load_gather(s3_v, [d])
            a2 = plsc.load_gather(s2_v, [t])
            ee = jnp.exp(a1 + a2 + a3)
            w_v[sl] = ee * nrm_v[sl]
            gidx_v[sl] = s * R + t
            plsc.addupdate_scatter(esum_v, [d >> 7, d & 127], ee)
        pltpu.async_copy(q_hbm.at[gidx_v], rows_g, sem).wait()

        def _scale(j, cc):
            wv = w_v[pl.ds(j * 16, 16)]
            for i in range(16):
                e = j * 16 + i
                w = wv[i]
                for k in range(OUT // 16):
                    rows_g[e, pl.ds(k * 16, 16)] = (
                        rows_g[e, pl.ds(k * 16, 16)] * w)
            return cc

        lax.fori_loop(0, C // 16, _scale, 0)
        pltpu.sync_copy(rows_g, h_sp.at[dst_v], add=True)
        return c

    lax.fori_loop(0, NCHUNK, _chunk, 0)
    plsc.subcore_barrier()

    def _out(k, c):
        sl = pl.ds(row0 + k * C, C)
        pltpu.sync_copy(h_sp.at[sl], h_out.at[cid, sl])
        return c

    lax.fori_loop(0, RPT // C, _out, 0)
    plsc.subcore_barrier()

    pltpu.sync_copy(esum_v, h_sp.at[pl.ds(sid * ESR, ESR)])
    plsc.subcore_barrier()

    @pl.when(sid < ESR // EST)
    def _reduce():
        for k in range(EST):
            for c in range(128 // 16):
                acc_v[k, pl.ds(c * 16, 16)] = zeros16
        for s in range(NS):
            pltpu.sync_copy(h_sp.at[pl.ds(s * ESR + sid * EST, EST)], tmp_v)
            for k in range(EST):
                for c in range(128 // 16):
                    sl = pl.ds(c * 16, 16)
                    acc_v[k, sl] = acc_v[k, sl] + tmp_v[k, sl]
        pltpu.sync_copy(acc_v, es_out.at[cid, pl.ds(sid * EST, EST)])


def _sc(src, dst, et, nrm, s1, s3, s2p, q_flat):
    mesh = plsc.VectorSubcoreMesh(core_axis_name="c", subcore_axis_name="s",
                                  num_cores=NC, num_subcores=NS)
    kern = pl.kernel(
        _sc_body,
        out_type=(
            jax.ShapeDtypeStruct((NC, NP, OUT), jnp.float32),
            jax.ShapeDtypeStruct((NC, ESR, 128), jnp.float32),
        ),
        mesh=mesh,
        compiler_params=pltpu.CompilerParams(needs_layout_passes=False),
        scratch_types=[
            pltpu.VMEM_SHARED((NP, OUT), jnp.float32),
            pltpu.VMEM((NP,), jnp.float32),
            pltpu.VMEM((NP,), jnp.float32),
            pltpu.VMEM((16,), jnp.float32),
            pltpu.VMEM((ESR, 128), jnp.float32),
            pltpu.VMEM((C,), jnp.int32),
            pltpu.VMEM((C,), jnp.int32),
            pltpu.VMEM((C,), jnp.int32),
            pltpu.VMEM((C,), jnp.float32),
            pltpu.VMEM((C,), jnp.int32),
            pltpu.VMEM((C,), jnp.float32),
            pltpu.VMEM((C,), jnp.float32),
            pltpu.VMEM((C, OUT), jnp.float32),
            pltpu.VMEM((EST, 128), jnp.float32),
            pltpu.VMEM((EST, 128), jnp.float32),
            pltpu.SemaphoreType.DMA,
        ],
    )
    return kern(src, dst, et, nrm, s1, s3, s2p, q_flat)


def _t2_body(hp_ref, es_ref, f_ref, lw_ref, g_ref, b_ref, hb_ref, y_ref):
    acc = hp_ref[0] + hp_ref[1]
    esum = es_ref[0] + es_ref[1]
    esum = jnp.where(esum > 0.0, esum, 1.0)
    h = acc / esum
    mu = jnp.mean(h, axis=1, keepdims=True)
    xc = h - mu
    var = jnp.mean(xc * xc, axis=1, keepdims=True)
    y = xc * lax.rsqrt(var + 1e-5) * g_ref[...] + b_ref[...] + hb_ref[...]
    y_ref[...] = y + jnp.dot(f_ref[...], lw_ref[...],
                             preferred_element_type=jnp.float32)


def _t2(h_parts, es_parts, feat_p, loop_weight, gamma, beta, hbias):
    nblk = NP // TBLK
    return pl.pallas_call(
        _t2_body,
        grid=(nblk,),
        in_specs=[
            pl.BlockSpec((NC, TBLK, OUT), lambda i: (0, i, 0)),
            pl.BlockSpec((NC, TBLK, 1), lambda i: (0, i, 0)),
            pl.BlockSpec((TBLK, IN), lambda i: (i, 0)),
            pl.BlockSpec((IN, OUT), lambda i: (0, 0)),
            pl.BlockSpec((1, OUT), lambda i: (0, 0)),
            pl.BlockSpec((1, OUT), lambda i: (0, 0)),
            pl.BlockSpec((1, OUT), lambda i: (0, 0)),
        ],
        out_specs=pl.BlockSpec((TBLK, OUT), lambda i: (i, 0)),
        out_shape=jax.ShapeDtypeStruct((NP, OUT), jnp.float32),
    )(h_parts, es_parts, feat_p, loop_weight, gamma, beta, hbias)


@jax.jit
def kernel(feat, edge_index, etypes, norm, bases, coeff, attn_weight, attn_vec,
           m_rel, h_bias, ln_gamma, ln_beta, loop_weight):
    src = edge_index[0].astype(jnp.int32)
    dst = edge_index[1].astype(jnp.int32)
    et = etypes.astype(jnp.int32)
    nrm = norm.reshape(E)
    feat_p = jnp.pad(feat, ((0, NP - N), (0, 0)))
    bases_flat = bases.reshape(B * IN, OUT)

    w2, u13, s2 = _t0(bases_flat, attn_weight, attn_vec, m_rel, coeff)
    q, s13 = _t1(feat_p, w2, u13)
    q_flat = q.reshape(NP * R, OUT)
    s1 = s13[:, 0]
    s3 = s13[:, 1]
    s2p = jnp.pad(s2.reshape(R), (0, 16 - R))

    h_parts, es_parts = _sc(src, dst, et, nrm, s1, s3, s2p, q_flat)

    y = _t2(h_parts, es_parts.reshape(NC, NP, 1), feat_p, loop_weight,
            ln_gamma.reshape(1, OUT), ln_beta.reshape(1, OUT),
            h_bias.reshape(1, OUT))
    return y[:N]
